# baseline (speedup 1.0000x reference)
import functools

import jax
import jax.numpy as jnp
from jax import lax
from jax.experimental import pallas as pl
from jax.experimental.pallas import tpu as pltpu
from jax.experimental.pallas import tpu_sc as plsc

T = 2048
C_IN = 2048
C_OUT = 2048
E = 8
BLK = 128
NPAD = 2 * T + E * BLK
NBLK = NPAD // BLK
COUT_T = 512
NCOL = C_OUT // COUT_T
ROW_T = 512

NW = 32
GCH = 32
TCH = 16


def _gate_body(x_ref, gw_ref, gb_ref, idx_ref, wts_ref):
    logits = lax.dot_general(
        x_ref[...], gw_ref[...], (((1,), (1,)), ((), ())),
        preferred_element_type=jnp.float32) + gb_ref[0, :][None, :]
    iota = lax.broadcasted_iota(jnp.int32, (ROW_T, E), 1)
    m1 = jnp.max(logits, axis=1, keepdims=True)
    i1 = jnp.min(jnp.where(logits == m1, iota, E), axis=1, keepdims=True)
    masked = jnp.where(iota == i1, -jnp.inf, logits)
    m2 = jnp.max(masked, axis=1, keepdims=True)
    i2 = jnp.min(jnp.where(masked == m2, iota, E), axis=1, keepdims=True)
    w1 = 1.0 / (1.0 + jnp.exp(m2 - m1))
    w2 = 1.0 - w1
    idx_ref[...] = jnp.where(iota == 0, i1, jnp.where(iota == 1, i2, 0))
    wts_ref[...] = jnp.where(iota == 0, w1, jnp.where(iota == 1, w2, 0.0))


def _gate(x_flat, gate_W, gate_b):
    idx, wts = pl.pallas_call(
        _gate_body,
        grid=(T // ROW_T,),
        in_specs=[
            pl.BlockSpec((ROW_T, C_IN), lambda i: (i, 0)),
            pl.BlockSpec((E, C_IN), lambda i: (0, 0)),
            pl.BlockSpec((1, E), lambda i: (0, 0)),
        ],
        out_specs=[
            pl.BlockSpec((ROW_T, E), lambda i: (i, 0)),
            pl.BlockSpec((ROW_T, E), lambda i: (i, 0)),
        ],
        out_shape=[
            jax.ShapeDtypeStruct((T, E), jnp.int32),
            jax.ShapeDtypeStruct((T, E), jnp.float32),
        ],
    )(x_flat, gate_W, gate_b.reshape(1, E))
    return idx[:, 0], idx[:, 1], wts[:, 0], wts[:, 1]


@functools.cache
def _sc_mesh():
    return plsc.VectorSubcoreMesh(core_axis_name="c", subcore_axis_name="s")


@functools.cache
def _make_sc_gather():
    @functools.partial(
        pl.kernel,
        out_type=jax.ShapeDtypeStruct((NPAD, C_IN), jnp.float32),
        mesh=_sc_mesh(),
        scratch_types=[
            pltpu.VMEM((GCH,), jnp.int32),
            pltpu.VMEM((GCH, C_IN), jnp.float32),
            pltpu.SemaphoreType.DMA,
        ],
    )
    def _sc_gather(x_hbm, tok_hbm, xs_hbm, idx_v, rows_v, sem):
        wid = lax.axis_index("s") * 2 + lax.axis_index("c")
        base = wid * (NPAD // NW)
        for ci in range(NPAD // NW // GCH):
            off = base + ci * GCH
            pltpu.sync_copy(tok_hbm.at[pl.ds(off, GCH)], idx_v)
            pltpu.async_copy(x_hbm.at[idx_v], rows_v, sem).wait()
            pltpu.sync_copy(rows_v, xs_hbm.at[pl.ds(off, GCH)])

    return _sc_gather


def _mm_body(be_ref, xs_ref, w_ref, wt_ref, ys_ref):
    acc = lax.dot_general(
        xs_ref[...], w_ref[0], (((1,), (1,)), ((), ())),
        preferred_element_type=jnp.float32)
    ys_ref[...] = acc * wt_ref[:, :1]


def _grouped_matmul(block_expert, xs, experts_W, w_bcast):
    grid_spec = pltpu.PrefetchScalarGridSpec(
        num_scalar_prefetch=1,
        grid=(NCOL, NBLK),
        in_specs=[
            pl.BlockSpec((BLK, C_IN), lambda j, i, be: (i, 0)),
            pl.BlockSpec((1, COUT_T, C_IN), lambda j, i, be: (be[i], j, 0)),
            pl.BlockSpec((BLK, 128), lambda j, i, be: (i, 0)),
        ],
        out_specs=pl.BlockSpec((BLK, COUT_T), lambda j, i, be: (i, j)),
    )
    return pl.pallas_call(
        _mm_body,
        grid_spec=grid_spec,
        out_shape=jax.ShapeDtypeStruct((NPAD, C_OUT), jnp.float32),
    )(block_expert, xs, experts_W, w_bcast)


@functools.cache
def _make_sc_combine():
    @functools.partial(
        pl.kernel,
        out_type=jax.ShapeDtypeStruct((T, C_OUT), jnp.float32),
        mesh=_sc_mesh(),
        scratch_types=[
            pltpu.VMEM((TCH,), jnp.int32),
            pltpu.VMEM((TCH,), jnp.int32),
            pltpu.VMEM((TCH, C_OUT), jnp.float32),
            pltpu.VMEM((TCH, C_OUT), jnp.float32),
            pltpu.SemaphoreType.DMA,
            pltpu.SemaphoreType.DMA,
        ],
    )
    def _sc_combine(ys_hbm, p1_hbm, p2_hbm, out_hbm, i1_v, i2_v, r1_v, r2_v,
                    sem1, sem2):
        wid = lax.axis_index("s") * 2 + lax.axis_index("c")
        base = wid * (T // NW)
        for ci in range(T // NW // TCH):
            off = base + ci * TCH
            pltpu.sync_copy(p1_hbm.at[pl.ds(off, TCH)], i1_v)
            pltpu.sync_copy(p2_hbm.at[pl.ds(off, TCH)], i2_v)
            c1 = pltpu.async_copy(ys_hbm.at[i1_v], r1_v, sem1)
            c2 = pltpu.async_copy(ys_hbm.at[i2_v], r2_v, sem2)
            c1.wait()
            c2.wait()
            for i in range(TCH):
                def body(j, _, i=i):
                    s = pl.ds(j * 16, 16)
                    r1_v[i, s] = r1_v[i, s] + r2_v[i, s]
                    return 0
                lax.fori_loop(0, C_OUT // 16, body, 0)
            pltpu.sync_copy(r1_v, out_hbm.at[pl.ds(off, TCH)])

    return _sc_combine


def _route_meta(i1, i2, w1, w2):
    e_all = jnp.concatenate([i1, i2])
    oh = (e_all[:, None] == jnp.arange(E)[None, :]).astype(jnp.int32)
    cum = jnp.cumsum(oh, axis=0)
    counts = cum[-1]
    rank = jnp.take_along_axis(cum, e_all[:, None], axis=1)[:, 0] - 1
    padded = ((counts + BLK - 1) // BLK) * BLK
    ends = jnp.cumsum(padded)
    p_all = (ends - padded)[e_all] + rank
    tok = jnp.concatenate([jnp.arange(T, dtype=jnp.int32)] * 2)
    slot_token = jnp.zeros((NPAD,), jnp.int32).at[p_all].set(tok)
    slot_w = jnp.zeros((NPAD,), jnp.float32).at[p_all].set(
        jnp.concatenate([w1, w2]))
    block_expert = jnp.clip(
        jnp.searchsorted(ends, jnp.arange(NBLK) * BLK, side="right"),
        0, E - 1).astype(jnp.int32)
    p1 = p_all[:T].astype(jnp.int32)
    p2 = p_all[T:].astype(jnp.int32)
    return slot_token, slot_w, block_expert, p1, p2


def kernel(x, experts_W, gate_W, gate_b):
    b, t, c_in = x.shape
    x_flat = x.reshape(t, c_in)
    i1, i2, w1, w2 = _gate(x_flat, gate_W, gate_b)
    slot_token, slot_w, block_expert, p1, p2 = _route_meta(i1, i2, w1, w2)
    w_bcast = jnp.broadcast_to(slot_w[:, None], (NPAD, 128))
    xs = _make_sc_gather()(x_flat, slot_token)
    ys = _grouped_matmul(block_expert, xs, experts_W, w_bcast)
    out = _make_sc_combine()(ys, p1, p2)
    return out.reshape(b, t, C_OUT)


# speedup vs baseline: 1.0048x; 1.0048x over previous
import functools

import jax
import jax.numpy as jnp
from jax import lax
from jax.experimental import pallas as pl
from jax.experimental.pallas import tpu as pltpu
from jax.experimental.pallas import tpu_sc as plsc

T = 2048
C_IN = 2048
C_OUT = 2048
E = 8
BLK = 128
NPAD = 2 * T + E * BLK
NBLK = NPAD // BLK
COUT_T = 512
NCOL = C_OUT // COUT_T
ROW_T = 512

NW = 32
GCH = 32
TCH = 16


def _gate_body(x_ref, gw_ref, gb_ref, idx_ref, wts_ref):
    logits = lax.dot_general(
        x_ref[...], gw_ref[...], (((1,), (1,)), ((), ())),
        preferred_element_type=jnp.float32) + gb_ref[0, :][None, :]
    iota = lax.broadcasted_iota(jnp.int32, (ROW_T, E), 1)
    m1 = jnp.max(logits, axis=1, keepdims=True)
    i1 = jnp.min(jnp.where(logits == m1, iota, E), axis=1, keepdims=True)
    masked = jnp.where(iota == i1, -jnp.inf, logits)
    m2 = jnp.max(masked, axis=1, keepdims=True)
    i2 = jnp.min(jnp.where(masked == m2, iota, E), axis=1, keepdims=True)
    w1 = 1.0 / (1.0 + jnp.exp(m2 - m1))
    w2 = 1.0 - w1
    idx_ref[...] = jnp.where(iota == 0, i1, jnp.where(iota == 1, i2, 0))
    wts_ref[...] = jnp.where(iota == 0, w1, jnp.where(iota == 1, w2, 0.0))


def _gate(x_flat, gate_W, gate_b):
    idx, wts = pl.pallas_call(
        _gate_body,
        grid=(T // ROW_T,),
        in_specs=[
            pl.BlockSpec((ROW_T, C_IN), lambda i: (i, 0)),
            pl.BlockSpec((E, C_IN), lambda i: (0, 0)),
            pl.BlockSpec((1, E), lambda i: (0, 0)),
        ],
        out_specs=[
            pl.BlockSpec((ROW_T, E), lambda i: (i, 0)),
            pl.BlockSpec((ROW_T, E), lambda i: (i, 0)),
        ],
        out_shape=[
            jax.ShapeDtypeStruct((T, E), jnp.int32),
            jax.ShapeDtypeStruct((T, E), jnp.float32),
        ],
    )(x_flat, gate_W, gate_b.reshape(1, E))
    return idx[:, 0], idx[:, 1], wts[:, 0], wts[:, 1]


@functools.cache
def _sc_mesh():
    return plsc.VectorSubcoreMesh(core_axis_name="c", subcore_axis_name="s")


NPAIR = 2 * T


@functools.cache
def _make_sc_gather():
    @functools.partial(
        pl.kernel,
        out_type=jax.ShapeDtypeStruct((NPAD, C_IN), jnp.float32),
        mesh=_sc_mesh(),
        scratch_types=[
            pltpu.VMEM((GCH,), jnp.int32),
            pltpu.VMEM((GCH,), jnp.int32),
            pltpu.VMEM((GCH, 16), jnp.float32),
            pltpu.VMEM((GCH, C_IN), jnp.float32),
            pltpu.SemaphoreType.DMA,
        ],
    )
    def _sc_gather(x_hbm, tok_hbm, pos_hbm, w_hbm, xs_hbm,
                   tok_v, pos_v, w_v, rows_v, sem):
        wid = lax.axis_index("s") * 2 + lax.axis_index("c")
        base = wid * (NPAIR // NW)
        for ci in range(NPAIR // NW // GCH):
            off = base + ci * GCH
            pltpu.sync_copy(tok_hbm.at[pl.ds(off, GCH)], tok_v)
            pltpu.sync_copy(pos_hbm.at[pl.ds(off, GCH)], pos_v)
            pltpu.sync_copy(w_hbm.at[pl.ds(off, GCH)], w_v)
            pltpu.async_copy(x_hbm.at[tok_v], rows_v, sem).wait()

            def scale_row(r, _):
                w_vec = w_v[r, :]

                def scale_vec(j, _):
                    s = pl.ds(j * 16, 16)
                    rows_v[r, s] = rows_v[r, s] * w_vec
                    return 0

                lax.fori_loop(0, C_IN // 16, scale_vec, 0)
                return 0

            lax.fori_loop(0, GCH, scale_row, 0)
            pltpu.async_copy(rows_v, xs_hbm.at[pos_v], sem).wait()

    return _sc_gather


def _mm_body(be_ref, xs_ref, w_ref, ys_ref):
    ys_ref[...] = lax.dot_general(
        xs_ref[...], w_ref[0], (((1,), (1,)), ((), ())),
        preferred_element_type=jnp.float32)


def _grouped_matmul(block_expert, xs, experts_W):
    grid_spec = pltpu.PrefetchScalarGridSpec(
        num_scalar_prefetch=1,
        grid=(NCOL, NBLK),
        in_specs=[
            pl.BlockSpec((BLK, C_IN), lambda j, i, be: (i, 0)),
            pl.BlockSpec((1, COUT_T, C_IN), lambda j, i, be: (be[i], j, 0)),
        ],
        out_specs=pl.BlockSpec((BLK, COUT_T), lambda j, i, be: (i, j)),
    )
    return pl.pallas_call(
        _mm_body,
        grid_spec=grid_spec,
        out_shape=jax.ShapeDtypeStruct((NPAD, C_OUT), jnp.float32),
    )(block_expert, xs, experts_W)


@functools.cache
def _make_sc_combine():
    @functools.partial(
        pl.kernel,
        out_type=jax.ShapeDtypeStruct((T, C_OUT), jnp.float32),
        mesh=_sc_mesh(),
        scratch_types=[
            pltpu.VMEM((TCH,), jnp.int32),
            pltpu.VMEM((TCH,), jnp.int32),
            pltpu.VMEM((TCH, C_OUT), jnp.float32),
            pltpu.VMEM((TCH, C_OUT), jnp.float32),
            pltpu.SemaphoreType.DMA,
            pltpu.SemaphoreType.DMA,
        ],
    )
    def _sc_combine(ys_hbm, p1_hbm, p2_hbm, out_hbm, i1_v, i2_v, r1_v, r2_v,
                    sem1, sem2):
        wid = lax.axis_index("s") * 2 + lax.axis_index("c")
        base = wid * (T // NW)
        for ci in range(T // NW // TCH):
            off = base + ci * TCH
            pltpu.sync_copy(p1_hbm.at[pl.ds(off, TCH)], i1_v)
            pltpu.sync_copy(p2_hbm.at[pl.ds(off, TCH)], i2_v)
            c1 = pltpu.async_copy(ys_hbm.at[i1_v], r1_v, sem1)
            c2 = pltpu.async_copy(ys_hbm.at[i2_v], r2_v, sem2)
            c1.wait()
            c2.wait()
            for i in range(TCH):
                def body(j, _, i=i):
                    s = pl.ds(j * 16, 16)
                    r1_v[i, s] = r1_v[i, s] + r2_v[i, s]
                    return 0
                lax.fori_loop(0, C_OUT // 16, body, 0)
            pltpu.sync_copy(r1_v, out_hbm.at[pl.ds(off, TCH)])

    return _sc_combine


def _route_meta(i1, i2):
    e_all = jnp.concatenate([i1, i2])
    oh = (e_all[:, None] == jnp.arange(E)[None, :]).astype(jnp.int32)
    cum = jnp.cumsum(oh, axis=0)
    counts = cum[-1]
    rank = jnp.take_along_axis(cum, e_all[:, None], axis=1)[:, 0] - 1
    padded = ((counts + BLK - 1) // BLK) * BLK
    ends = jnp.cumsum(padded)
    p_all = ((ends - padded)[e_all] + rank).astype(jnp.int32)
    block_expert = jnp.clip(
        jnp.searchsorted(ends, jnp.arange(NBLK) * BLK, side="right"),
        0, E - 1).astype(jnp.int32)
    return p_all, block_expert


def kernel(x, experts_W, gate_W, gate_b):
    b, t, c_in = x.shape
    x_flat = x.reshape(t, c_in)
    i1, i2, w1, w2 = _gate(x_flat, gate_W, gate_b)
    p_all, block_expert = _route_meta(i1, i2)
    tok_all = jnp.concatenate([jnp.arange(T, dtype=jnp.int32)] * 2)
    w_all = jnp.broadcast_to(
        jnp.concatenate([w1, w2])[:, None], (NPAIR, 16))
    xs = _make_sc_gather()(x_flat, tok_all, p_all, w_all)
    ys = _grouped_matmul(block_expert, xs, experts_W)
    out = _make_sc_combine()(ys, p_all[:T], p_all[T:])
    return out.reshape(b, t, C_OUT)


# speedup vs baseline: 1.1647x; 1.1592x over previous
import functools

import jax
import jax.numpy as jnp
from jax import lax
from jax.experimental import pallas as pl
from jax.experimental.pallas import tpu as pltpu
from jax.experimental.pallas import tpu_sc as plsc

T = 2048
C_IN = 2048
C_OUT = 2048
E = 8
BLK = 256
NPAD = 2 * T + E * BLK
NBLK = NPAD // BLK
COUT_T = 512
NCOL = C_OUT // COUT_T
ROW_T = 512

NW = 32
GCH = 32
TCH = 16


def _gate_body(x_ref, gw_ref, gb_ref, idx_ref, wts_ref):
    logits = lax.dot_general(
        x_ref[...], gw_ref[...], (((1,), (1,)), ((), ())),
        preferred_element_type=jnp.float32) + gb_ref[0, :][None, :]
    iota = lax.broadcasted_iota(jnp.int32, (ROW_T, E), 1)
    m1 = jnp.max(logits, axis=1, keepdims=True)
    i1 = jnp.min(jnp.where(logits == m1, iota, E), axis=1, keepdims=True)
    masked = jnp.where(iota == i1, -jnp.inf, logits)
    m2 = jnp.max(masked, axis=1, keepdims=True)
    i2 = jnp.min(jnp.where(masked == m2, iota, E), axis=1, keepdims=True)
    w1 = 1.0 / (1.0 + jnp.exp(m2 - m1))
    w2 = 1.0 - w1
    idx_ref[...] = jnp.where(iota == 0, i1, jnp.where(iota == 1, i2, 0))
    wts_ref[...] = jnp.where(iota == 0, w1, jnp.where(iota == 1, w2, 0.0))


def _gate(x_flat, gate_W, gate_b):
    idx, wts = pl.pallas_call(
        _gate_body,
        grid=(T // ROW_T,),
        in_specs=[
            pl.BlockSpec((ROW_T, C_IN), lambda i: (i, 0)),
            pl.BlockSpec((E, C_IN), lambda i: (0, 0)),
            pl.BlockSpec((1, E), lambda i: (0, 0)),
        ],
        out_specs=[
            pl.BlockSpec((ROW_T, E), lambda i: (i, 0)),
            pl.BlockSpec((ROW_T, E), lambda i: (i, 0)),
        ],
        out_shape=[
            jax.ShapeDtypeStruct((T, E), jnp.int32),
            jax.ShapeDtypeStruct((T, E), jnp.float32),
        ],
    )(x_flat, gate_W, gate_b.reshape(1, E))
    return idx[:, 0], idx[:, 1], wts[:, 0], wts[:, 1]


@functools.cache
def _sc_mesh():
    return plsc.VectorSubcoreMesh(core_axis_name="c", subcore_axis_name="s")


NPAIR = 2 * T


@functools.cache
def _make_sc_gather():
    @functools.partial(
        pl.kernel,
        out_type=jax.ShapeDtypeStruct((NPAD, C_IN), jnp.float32),
        mesh=_sc_mesh(),
        scratch_types=[
            pltpu.VMEM((GCH,), jnp.int32),
            pltpu.VMEM((GCH,), jnp.int32),
            pltpu.VMEM((GCH, 16), jnp.float32),
            pltpu.VMEM((GCH, C_IN), jnp.float32),
            pltpu.SemaphoreType.DMA,
        ],
    )
    def _sc_gather(x_hbm, tok_hbm, pos_hbm, w_hbm, xs_hbm,
                   tok_v, pos_v, w_v, rows_v, sem):
        wid = lax.axis_index("s") * 2 + lax.axis_index("c")
        base = wid * (NPAIR // NW)
        for ci in range(NPAIR // NW // GCH):
            off = base + ci * GCH
            pltpu.sync_copy(tok_hbm.at[pl.ds(off, GCH)], tok_v)
            pltpu.sync_copy(pos_hbm.at[pl.ds(off, GCH)], pos_v)
            pltpu.sync_copy(w_hbm.at[pl.ds(off, GCH)], w_v)
            pltpu.async_copy(x_hbm.at[tok_v], rows_v, sem).wait()

            def scale_row(r, _):
                w_vec = w_v[r, :]

                def scale_vec(j, _):
                    s = pl.ds(j * 16, 16)
                    rows_v[r, s] = rows_v[r, s] * w_vec
                    return 0

                lax.fori_loop(0, C_IN // 16, scale_vec, 0)
                return 0

            lax.fori_loop(0, GCH, scale_row, 0)
            pltpu.async_copy(rows_v, xs_hbm.at[pos_v], sem).wait()

    return _sc_gather


def _mm_body(be_ref, xs_ref, w_ref, ys_ref):
    ys_ref[...] = lax.dot_general(
        xs_ref[...], w_ref[0], (((1,), (1,)), ((), ())),
        preferred_element_type=jnp.float32)


def _grouped_matmul(block_expert, xs, experts_W):
    grid_spec = pltpu.PrefetchScalarGridSpec(
        num_scalar_prefetch=1,
        grid=(NCOL, NBLK),
        in_specs=[
            pl.BlockSpec((BLK, C_IN), lambda j, i, be: (i, 0)),
            pl.BlockSpec((1, COUT_T, C_IN), lambda j, i, be: (be[i], j, 0)),
        ],
        out_specs=pl.BlockSpec((BLK, COUT_T), lambda j, i, be: (i, j)),
    )
    return pl.pallas_call(
        _mm_body,
        grid_spec=grid_spec,
        out_shape=jax.ShapeDtypeStruct((NPAD, C_OUT), jnp.float32),
    )(block_expert, xs, experts_W)


@functools.cache
def _make_sc_combine():
    @functools.partial(
        pl.kernel,
        out_type=jax.ShapeDtypeStruct((T, C_OUT), jnp.float32),
        mesh=_sc_mesh(),
        scratch_types=[
            pltpu.VMEM((TCH,), jnp.int32),
            pltpu.VMEM((TCH,), jnp.int32),
            pltpu.VMEM((TCH, C_OUT), jnp.float32),
            pltpu.VMEM((TCH, C_OUT), jnp.float32),
            pltpu.SemaphoreType.DMA,
            pltpu.SemaphoreType.DMA,
        ],
    )
    def _sc_combine(ys_hbm, p1_hbm, p2_hbm, out_hbm, i1_v, i2_v, r1_v, r2_v,
                    sem1, sem2):
        wid = lax.axis_index("s") * 2 + lax.axis_index("c")
        base = wid * (T // NW)
        for ci in range(T // NW // TCH):
            off = base + ci * TCH
            pltpu.sync_copy(p1_hbm.at[pl.ds(off, TCH)], i1_v)
            pltpu.sync_copy(p2_hbm.at[pl.ds(off, TCH)], i2_v)
            c1 = pltpu.async_copy(ys_hbm.at[i1_v], r1_v, sem1)
            c2 = pltpu.async_copy(ys_hbm.at[i2_v], r2_v, sem2)
            c1.wait()
            c2.wait()
            for i in range(TCH):
                def body(j, _, i=i):
                    s = pl.ds(j * 16, 16)
                    r1_v[i, s] = r1_v[i, s] + r2_v[i, s]
                    return 0
                lax.fori_loop(0, C_OUT // 16, body, 0)
            pltpu.sync_copy(r1_v, out_hbm.at[pl.ds(off, TCH)])

    return _sc_combine


def _route_meta(i1, i2):
    e_all = jnp.concatenate([i1, i2])
    oh = (e_all[:, None] == jnp.arange(E)[None, :]).astype(jnp.int32)
    cum = jnp.cumsum(oh, axis=0)
    counts = cum[-1]
    rank = jnp.take_along_axis(cum, e_all[:, None], axis=1)[:, 0] - 1
    padded = ((counts + BLK - 1) // BLK) * BLK
    ends = jnp.cumsum(padded)
    p_all = ((ends - padded)[e_all] + rank).astype(jnp.int32)
    block_expert = jnp.clip(
        jnp.searchsorted(ends, jnp.arange(NBLK) * BLK, side="right"),
        0, E - 1).astype(jnp.int32)
    return p_all, block_expert


def kernel(x, experts_W, gate_W, gate_b):
    b, t, c_in = x.shape
    x_flat = x.reshape(t, c_in)
    i1, i2, w1, w2 = _gate(x_flat, gate_W, gate_b)
    p_all, block_expert = _route_meta(i1, i2)
    tok_all = jnp.concatenate([jnp.arange(T, dtype=jnp.int32)] * 2)
    w_all = jnp.broadcast_to(
        jnp.concatenate([w1, w2])[:, None], (NPAIR, 16))
    xs = _make_sc_gather()(x_flat, tok_all, p_all, w_all)
    ys = _grouped_matmul(block_expert, xs, experts_W)
    out = _make_sc_combine()(ys, p_all[:T], p_all[T:])
    return out.reshape(b, t, C_OUT)


# speedup vs baseline: 1.1718x; 1.0061x over previous
import functools

import jax
import jax.numpy as jnp
from jax import lax
from jax.experimental import pallas as pl
from jax.experimental.pallas import tpu as pltpu
from jax.experimental.pallas import tpu_sc as plsc

T = 2048
C_IN = 2048
C_OUT = 2048
E = 8
BLK = 256
NPAD = 2 * T + E * BLK
NBLK = NPAD // BLK
COUT_T = 512
NCOL = C_OUT // COUT_T
ROW_T = 512

NW = 32
GCH = 32
TCH = 16


def _gate_body(x_ref, gw_ref, gb_ref, idx_ref, wts_ref):
    logits = lax.dot_general(
        x_ref[...], gw_ref[...], (((1,), (1,)), ((), ())),
        preferred_element_type=jnp.float32) + gb_ref[0, :][None, :]
    iota = lax.broadcasted_iota(jnp.int32, (ROW_T, E), 1)
    m1 = jnp.max(logits, axis=1, keepdims=True)
    i1 = jnp.min(jnp.where(logits == m1, iota, E), axis=1, keepdims=True)
    masked = jnp.where(iota == i1, -jnp.inf, logits)
    m2 = jnp.max(masked, axis=1, keepdims=True)
    i2 = jnp.min(jnp.where(masked == m2, iota, E), axis=1, keepdims=True)
    w1 = 1.0 / (1.0 + jnp.exp(m2 - m1))
    w2 = 1.0 - w1
    idx_ref[...] = jnp.where(iota == 0, i1, jnp.where(iota == 1, i2, 0))
    wts_ref[...] = jnp.where(iota == 0, w1, jnp.where(iota == 1, w2, 0.0))


def _gate(x_flat, gate_W, gate_b):
    idx, wts = pl.pallas_call(
        _gate_body,
        grid=(T // ROW_T,),
        in_specs=[
            pl.BlockSpec((ROW_T, C_IN), lambda i: (i, 0)),
            pl.BlockSpec((E, C_IN), lambda i: (0, 0)),
            pl.BlockSpec((1, E), lambda i: (0, 0)),
        ],
        out_specs=[
            pl.BlockSpec((ROW_T, E), lambda i: (i, 0)),
            pl.BlockSpec((ROW_T, E), lambda i: (i, 0)),
        ],
        out_shape=[
            jax.ShapeDtypeStruct((T, E), jnp.int32),
            jax.ShapeDtypeStruct((T, E), jnp.float32),
        ],
    )(x_flat, gate_W, gate_b.reshape(1, E))
    return idx[:, 0], idx[:, 1], wts[:, 0], wts[:, 1]


@functools.cache
def _sc_mesh():
    return plsc.VectorSubcoreMesh(core_axis_name="c", subcore_axis_name="s")


NPAIR = 2 * T


@functools.cache
def _make_sc_gather():
    @functools.partial(
        pl.kernel,
        out_type=jax.ShapeDtypeStruct((NPAD, C_IN), jnp.float32),
        mesh=_sc_mesh(),
        scratch_types=[
            pltpu.VMEM((GCH,), jnp.int32),
            pltpu.VMEM((GCH,), jnp.int32),
            pltpu.VMEM((GCH, 16), jnp.float32),
            pltpu.VMEM((GCH, C_IN), jnp.float32),
            pltpu.SemaphoreType.DMA,
        ],
    )
    def _sc_gather(x_hbm, tok_hbm, pos_hbm, w_hbm, xs_hbm,
                   tok_v, pos_v, w_v, rows_v, sem):
        wid = lax.axis_index("s") * 2 + lax.axis_index("c")
        base = wid * (NPAIR // NW)
        for ci in range(NPAIR // NW // GCH):
            off = base + ci * GCH
            pltpu.sync_copy(tok_hbm.at[pl.ds(off, GCH)], tok_v)
            pltpu.sync_copy(pos_hbm.at[pl.ds(off, GCH)], pos_v)
            pltpu.sync_copy(w_hbm.at[pl.ds(off, GCH)], w_v)
            pltpu.async_copy(x_hbm.at[tok_v], rows_v, sem).wait()

            def scale_row(r, _):
                w_vec = w_v[r, :]

                def scale_vec(j, _):
                    s = pl.ds(j * 16, 16)
                    rows_v[r, s] = rows_v[r, s] * w_vec
                    return 0

                lax.fori_loop(0, C_IN // 16, scale_vec, 0)
                return 0

            lax.fori_loop(0, GCH, scale_row, 0)
            pltpu.async_copy(rows_v, xs_hbm.at[pos_v], sem).wait()

    return _sc_gather


def _mm_body(be_ref, xs_ref, w_ref, ys_ref):
    ys_ref[...] = lax.dot_general(
        xs_ref[...], w_ref[0], (((1,), (1,)), ((), ())),
        preferred_element_type=jnp.float32)


def _grouped_matmul(block_expert, xs, experts_W):
    grid_spec = pltpu.PrefetchScalarGridSpec(
        num_scalar_prefetch=1,
        grid=(NCOL, NBLK),
        in_specs=[
            pl.BlockSpec((BLK, C_IN), lambda j, i, be: (i, 0)),
            pl.BlockSpec((1, COUT_T, C_IN), lambda j, i, be: (be[i], j, 0)),
        ],
        out_specs=pl.BlockSpec((BLK, COUT_T), lambda j, i, be: (i, j)),
    )
    return pl.pallas_call(
        _mm_body,
        grid_spec=grid_spec,
        out_shape=jax.ShapeDtypeStruct((NPAD, C_OUT), jnp.float32),
    )(block_expert, xs, experts_W)


@functools.cache
def _make_sc_combine():
    @functools.partial(
        pl.kernel,
        out_type=jax.ShapeDtypeStruct((T, C_OUT), jnp.float32),
        mesh=_sc_mesh(),
        scratch_types=[
            pltpu.VMEM((TCH,), jnp.int32),
            pltpu.VMEM((TCH,), jnp.int32),
            pltpu.VMEM((TCH, C_OUT), jnp.float32),
            pltpu.VMEM((TCH, C_OUT), jnp.float32),
            pltpu.SemaphoreType.DMA,
            pltpu.SemaphoreType.DMA,
        ],
    )
    def _sc_combine(ys_hbm, p1_hbm, p2_hbm, out_hbm, i1_v, i2_v, r1_v, r2_v,
                    sem1, sem2):
        wid = lax.axis_index("s") * 2 + lax.axis_index("c")
        base = wid * (T // NW)
        for ci in range(T // NW // TCH):
            off = base + ci * TCH
            pltpu.sync_copy(p1_hbm.at[pl.ds(off, TCH)], i1_v)
            pltpu.sync_copy(p2_hbm.at[pl.ds(off, TCH)], i2_v)
            c1 = pltpu.async_copy(ys_hbm.at[i1_v], r1_v, sem1)
            c2 = pltpu.async_copy(ys_hbm.at[i2_v], r2_v, sem2)
            c1.wait()
            c2.wait()
            for i in range(TCH):
                def body(j, _, i=i):
                    s = pl.ds(j * 16, 16)
                    r1_v[i, s] = r1_v[i, s] + r2_v[i, s]
                    return 0
                lax.fori_loop(0, C_OUT // 16, body, 0)
            pltpu.sync_copy(r1_v, out_hbm.at[pl.ds(off, TCH)])

    return _sc_combine


def _route_meta(i1, i2):
    e_all = jnp.concatenate([i1, i2])
    oh = (e_all[:, None] == jnp.arange(E)[None, :]).astype(jnp.float32)
    CB = 128
    NB = NPAIR // CB
    ohb = oh.reshape(NB, CB, E)
    tri = (jnp.arange(CB)[:, None] >= jnp.arange(CB)[None, :]).astype(
        jnp.float32)
    incl = jnp.einsum("lk,bke->ble", tri, ohb)
    bsum = incl[:, -1, :]
    bpre = jnp.cumsum(bsum, axis=0) - bsum
    cum = (incl + bpre[:, None, :]).reshape(NPAIR, E).astype(jnp.int32)
    counts = cum[-1]
    rank = jnp.take_along_axis(cum, e_all[:, None], axis=1)[:, 0] - 1
    padded = ((counts + BLK - 1) // BLK) * BLK
    ends = jnp.cumsum(padded)
    p_all = ((ends - padded)[e_all] + rank).astype(jnp.int32)
    block_expert = jnp.clip(
        jnp.searchsorted(ends, jnp.arange(NBLK) * BLK, side="right"),
        0, E - 1).astype(jnp.int32)
    return p_all, block_expert


def kernel(x, experts_W, gate_W, gate_b):
    b, t, c_in = x.shape
    x_flat = x.reshape(t, c_in)
    i1, i2, w1, w2 = _gate(x_flat, gate_W, gate_b)
    p_all, block_expert = _route_meta(i1, i2)
    tok_all = jnp.concatenate([jnp.arange(T, dtype=jnp.int32)] * 2)
    w_all = jnp.broadcast_to(
        jnp.concatenate([w1, w2])[:, None], (NPAIR, 16))
    xs = _make_sc_gather()(x_flat, tok_all, p_all, w_all)
    ys = _grouped_matmul(block_expert, xs, experts_W)
    out = _make_sc_combine()(ys, p_all[:T], p_all[T:])
    return out.reshape(b, t, C_OUT)


# speedup vs baseline: 1.4139x; 1.2065x over previous
import functools

import jax
import jax.numpy as jnp
from jax import lax
from jax.experimental import pallas as pl
from jax.experimental.pallas import tpu as pltpu
from jax.experimental.pallas import tpu_sc as plsc

T = 2048
C_IN = 2048
C_OUT = 2048
E = 8
BLK = 256
NPAD = 2 * T + E * BLK
NBLK = NPAD // BLK
COUT_T = 2048
NCOL = C_OUT // COUT_T
ROW_T = 512

NW = 32
GCH = 32
TCH = 16


def _gate_body(x_ref, gw_ref, gb_ref, idx_ref, wts_ref):
    logits = lax.dot_general(
        x_ref[...], gw_ref[...], (((1,), (1,)), ((), ())),
        preferred_element_type=jnp.float32) + gb_ref[0, :][None, :]
    iota = lax.broadcasted_iota(jnp.int32, (ROW_T, E), 1)
    m1 = jnp.max(logits, axis=1, keepdims=True)
    i1 = jnp.min(jnp.where(logits == m1, iota, E), axis=1, keepdims=True)
    masked = jnp.where(iota == i1, -jnp.inf, logits)
    m2 = jnp.max(masked, axis=1, keepdims=True)
    i2 = jnp.min(jnp.where(masked == m2, iota, E), axis=1, keepdims=True)
    w1 = 1.0 / (1.0 + jnp.exp(m2 - m1))
    w2 = 1.0 - w1
    idx_ref[...] = jnp.where(iota == 0, i1, jnp.where(iota == 1, i2, 0))
    wts_ref[...] = jnp.where(iota == 0, w1, jnp.where(iota == 1, w2, 0.0))


def _gate(x_flat, gate_W, gate_b):
    idx, wts = pl.pallas_call(
        _gate_body,
        grid=(T // ROW_T,),
        in_specs=[
            pl.BlockSpec((ROW_T, C_IN), lambda i: (i, 0)),
            pl.BlockSpec((E, C_IN), lambda i: (0, 0)),
            pl.BlockSpec((1, E), lambda i: (0, 0)),
        ],
        out_specs=[
            pl.BlockSpec((ROW_T, E), lambda i: (i, 0)),
            pl.BlockSpec((ROW_T, E), lambda i: (i, 0)),
        ],
        out_shape=[
            jax.ShapeDtypeStruct((T, E), jnp.int32),
            jax.ShapeDtypeStruct((T, E), jnp.float32),
        ],
    )(x_flat, gate_W, gate_b.reshape(1, E))
    return idx[:, 0], idx[:, 1], wts[:, 0], wts[:, 1]


@functools.cache
def _sc_mesh():
    return plsc.VectorSubcoreMesh(core_axis_name="c", subcore_axis_name="s")


NPAIR = 2 * T


@functools.cache
def _make_sc_gather():
    @functools.partial(
        pl.kernel,
        out_type=jax.ShapeDtypeStruct((NPAD, C_IN), jnp.float32),
        mesh=_sc_mesh(),
        scratch_types=[
            pltpu.VMEM((GCH,), jnp.int32),
            pltpu.VMEM((GCH,), jnp.int32),
            pltpu.VMEM((GCH, 16), jnp.float32),
            pltpu.VMEM((GCH, C_IN), jnp.float32),
            pltpu.SemaphoreType.DMA,
        ],
    )
    def _sc_gather(x_hbm, tok_hbm, pos_hbm, w_hbm, xs_hbm,
                   tok_v, pos_v, w_v, rows_v, sem):
        wid = lax.axis_index("s") * 2 + lax.axis_index("c")
        base = wid * (NPAIR // NW)
        for ci in range(NPAIR // NW // GCH):
            off = base + ci * GCH
            pltpu.sync_copy(tok_hbm.at[pl.ds(off, GCH)], tok_v)
            pltpu.sync_copy(pos_hbm.at[pl.ds(off, GCH)], pos_v)
            pltpu.sync_copy(w_hbm.at[pl.ds(off, GCH)], w_v)
            pltpu.async_copy(x_hbm.at[tok_v], rows_v, sem).wait()

            def scale_row(r, _):
                w_vec = w_v[r, :]

                def scale_vec(j, _):
                    s = pl.ds(j * 16, 16)
                    rows_v[r, s] = rows_v[r, s] * w_vec
                    return 0

                lax.fori_loop(0, C_IN // 16, scale_vec, 0)
                return 0

            lax.fori_loop(0, GCH, scale_row, 0)
            pltpu.async_copy(rows_v, xs_hbm.at[pos_v], sem).wait()

    return _sc_gather


def _mm_body(be_ref, xs_ref, w_ref, ys_ref):
    ys_ref[...] = lax.dot_general(
        xs_ref[...], w_ref[0], (((1,), (1,)), ((), ())),
        preferred_element_type=jnp.float32)


def _grouped_matmul(block_expert, xs, experts_W):
    grid_spec = pltpu.PrefetchScalarGridSpec(
        num_scalar_prefetch=1,
        grid=(NCOL, NBLK),
        in_specs=[
            pl.BlockSpec((BLK, C_IN), lambda j, i, be: (i, 0)),
            pl.BlockSpec((1, COUT_T, C_IN), lambda j, i, be: (be[i], j, 0)),
        ],
        out_specs=pl.BlockSpec((BLK, COUT_T), lambda j, i, be: (i, j)),
    )
    return pl.pallas_call(
        _mm_body,
        grid_spec=grid_spec,
        out_shape=jax.ShapeDtypeStruct((NPAD, C_OUT), jnp.float32),
    )(block_expert, xs, experts_W)


@functools.cache
def _make_sc_combine():
    @functools.partial(
        pl.kernel,
        out_type=jax.ShapeDtypeStruct((T, C_OUT), jnp.float32),
        mesh=_sc_mesh(),
        scratch_types=[
            pltpu.VMEM((TCH,), jnp.int32),
            pltpu.VMEM((TCH,), jnp.int32),
            pltpu.VMEM((TCH, C_OUT), jnp.float32),
            pltpu.VMEM((TCH, C_OUT), jnp.float32),
            pltpu.SemaphoreType.DMA,
            pltpu.SemaphoreType.DMA,
        ],
    )
    def _sc_combine(ys_hbm, p1_hbm, p2_hbm, out_hbm, i1_v, i2_v, r1_v, r2_v,
                    sem1, sem2):
        wid = lax.axis_index("s") * 2 + lax.axis_index("c")
        base = wid * (T // NW)
        for ci in range(T // NW // TCH):
            off = base + ci * TCH
            pltpu.sync_copy(p1_hbm.at[pl.ds(off, TCH)], i1_v)
            pltpu.sync_copy(p2_hbm.at[pl.ds(off, TCH)], i2_v)
            c1 = pltpu.async_copy(ys_hbm.at[i1_v], r1_v, sem1)
            c2 = pltpu.async_copy(ys_hbm.at[i2_v], r2_v, sem2)
            c1.wait()
            c2.wait()
            for i in range(TCH):
                def body(j, _, i=i):
                    s = pl.ds(j * 16, 16)
                    r1_v[i, s] = r1_v[i, s] + r2_v[i, s]
                    return 0
                lax.fori_loop(0, C_OUT // 16, body, 0)
            pltpu.sync_copy(r1_v, out_hbm.at[pl.ds(off, TCH)])

    return _sc_combine


def _route_meta(i1, i2):
    e_all = jnp.concatenate([i1, i2])
    oh = (e_all[:, None] == jnp.arange(E)[None, :]).astype(jnp.float32)
    CB = 128
    NB = NPAIR // CB
    ohb = oh.reshape(NB, CB, E)
    tri = (jnp.arange(CB)[:, None] >= jnp.arange(CB)[None, :]).astype(
        jnp.float32)
    incl = jnp.einsum("lk,bke->ble", tri, ohb)
    bsum = incl[:, -1, :]
    bpre = jnp.cumsum(bsum, axis=0) - bsum
    cum = (incl + bpre[:, None, :]).reshape(NPAIR, E).astype(jnp.int32)
    counts = cum[-1]
    ohi = oh.astype(jnp.int32)
    rank = jnp.sum(ohi * cum, axis=1) - 1
    padded = ((counts + BLK - 1) // BLK) * BLK
    ends = jnp.cumsum(padded)
    starts = jnp.sum(ohi * (ends - padded)[None, :], axis=1)
    p_all = (starts + rank).astype(jnp.int32)
    block_expert = jnp.clip(
        jnp.searchsorted(ends, jnp.arange(NBLK) * BLK, side="right"),
        0, E - 1).astype(jnp.int32)
    return p_all, block_expert


def kernel(x, experts_W, gate_W, gate_b):
    b, t, c_in = x.shape
    x_flat = x.reshape(t, c_in)
    i1, i2, w1, w2 = _gate(x_flat, gate_W, gate_b)
    p_all, block_expert = _route_meta(i1, i2)
    tok_all = jnp.concatenate([jnp.arange(T, dtype=jnp.int32)] * 2)
    w_all = jnp.broadcast_to(
        jnp.concatenate([w1, w2])[:, None], (NPAIR, 16))
    xs = _make_sc_gather()(x_flat, tok_all, p_all, w_all)
    ys = _grouped_matmul(block_expert, xs, experts_W)
    out = _make_sc_combine()(ys, p_all[:T], p_all[T:])
    return out.reshape(b, t, C_OUT)


# speedup vs baseline: 1.9623x; 1.3879x over previous
import functools

import jax
import jax.numpy as jnp
from jax import lax
from jax.experimental import pallas as pl
from jax.experimental.pallas import tpu as pltpu
from jax.experimental.pallas import tpu_sc as plsc

T = 2048
C_IN = 2048
C_OUT = 2048
E = 8
BLK = 256
NPAD = 2 * T + E * BLK
NBLK = NPAD // BLK
COUT_T = 2048
NCOL = C_OUT // COUT_T
ROW_T = 512

NW = 32
GCH = 32
TCH = 16


def _gate_body(x_ref, gw_ref, gb_ref, idx_ref, wts_ref):
    logits = lax.dot_general(
        x_ref[...], gw_ref[...], (((1,), (1,)), ((), ())),
        preferred_element_type=jnp.float32) + gb_ref[0, :][None, :]
    iota = lax.broadcasted_iota(jnp.int32, (ROW_T, E), 1)
    m1 = jnp.max(logits, axis=1, keepdims=True)
    i1 = jnp.min(jnp.where(logits == m1, iota, E), axis=1, keepdims=True)
    masked = jnp.where(iota == i1, -jnp.inf, logits)
    m2 = jnp.max(masked, axis=1, keepdims=True)
    i2 = jnp.min(jnp.where(masked == m2, iota, E), axis=1, keepdims=True)
    w1 = 1.0 / (1.0 + jnp.exp(m2 - m1))
    w2 = 1.0 - w1
    idx_ref[...] = jnp.where(iota == 0, i1, jnp.where(iota == 1, i2, 0))
    wts_ref[...] = jnp.where(iota == 0, w1, jnp.where(iota == 1, w2, 0.0))


def _gate(x_flat, gate_W, gate_b):
    idx, wts = pl.pallas_call(
        _gate_body,
        grid=(T // ROW_T,),
        in_specs=[
            pl.BlockSpec((ROW_T, C_IN), lambda i: (i, 0)),
            pl.BlockSpec((E, C_IN), lambda i: (0, 0)),
            pl.BlockSpec((1, E), lambda i: (0, 0)),
        ],
        out_specs=[
            pl.BlockSpec((ROW_T, E), lambda i: (i, 0)),
            pl.BlockSpec((ROW_T, E), lambda i: (i, 0)),
        ],
        out_shape=[
            jax.ShapeDtypeStruct((T, E), jnp.int32),
            jax.ShapeDtypeStruct((T, E), jnp.float32),
        ],
    )(x_flat, gate_W, gate_b.reshape(1, E))
    return idx[:, 0], idx[:, 1], wts[:, 0], wts[:, 1]


@functools.cache
def _sc_mesh():
    return plsc.VectorSubcoreMesh(core_axis_name="c", subcore_axis_name="s")


NPAIR = 2 * T


GCH2 = 16
NCHUNK = NPAIR // NW // GCH2


@functools.cache
def _make_sc_gather():
    @functools.partial(
        pl.kernel,
        out_type=jax.ShapeDtypeStruct((NPAD, C_IN), jnp.float32),
        mesh=_sc_mesh(),
        scratch_types=[
            pltpu.VMEM((NPAIR // NW,), jnp.int32),
            pltpu.VMEM((NCHUNK, GCH2), jnp.int32),
            pltpu.VMEM((GCH2, C_IN), jnp.float32),
            pltpu.VMEM((GCH2, C_IN), jnp.float32),
            pltpu.SemaphoreType.DMA,
            pltpu.SemaphoreType.DMA,
            pltpu.SemaphoreType.DMA,
            pltpu.SemaphoreType.DMA,
        ],
    )
    def _sc_gather(x_hbm, tok_hbm, pos_hbm, xs_hbm,
                   tok_v, pos_v, buf0, buf1, g0, g1, s0, s1):
        wid = lax.axis_index("s") * 2 + lax.axis_index("c")
        base = wid * (NPAIR // NW)
        pltpu.sync_copy(tok_hbm.at[pl.ds(base, NPAIR // NW)], tok_v)
        pltpu.sync_copy(pos_hbm.at[wid], pos_v)
        bufs = (buf0, buf1)
        gsems = (g0, g1)
        ssems = (s0, s1)

        def gather(ci):
            cp = pltpu.make_async_copy(
                x_hbm.at[tok_v.at[pl.ds(ci * GCH2, GCH2)]],
                bufs[ci % 2], gsems[ci % 2])
            cp.start()
            return cp

        def scatter(ci):
            cp = pltpu.make_async_copy(
                bufs[ci % 2], xs_hbm.at[pos_v.at[ci]], ssems[ci % 2])
            cp.start()
            return cp

        pend_g = {0: gather(0)}
        pend_s = {}
        for ci in range(NCHUNK):
            nxt = ci + 1
            if nxt < NCHUNK:
                if nxt - 2 >= 0:
                    pend_s.pop(nxt - 2).wait()
                pend_g[nxt] = gather(nxt)
            pend_g.pop(ci).wait()
            pend_s[ci] = scatter(ci)
        pend_s.pop(NCHUNK - 2).wait()
        pend_s.pop(NCHUNK - 1).wait()

    return _sc_gather


def _mm_body(be_ref, xs_ref, w_ref, wt_ref, ys_ref):
    acc = lax.dot_general(
        xs_ref[...], w_ref[0], (((1,), (1,)), ((), ())),
        preferred_element_type=jnp.float32)
    ys_ref[...] = acc * wt_ref[:, :1]


def _grouped_matmul(block_expert, xs, experts_W, w_bcast):
    grid_spec = pltpu.PrefetchScalarGridSpec(
        num_scalar_prefetch=1,
        grid=(NCOL, NBLK),
        in_specs=[
            pl.BlockSpec((BLK, C_IN), lambda j, i, be: (i, 0)),
            pl.BlockSpec((1, COUT_T, C_IN), lambda j, i, be: (be[i], j, 0)),
            pl.BlockSpec((BLK, 128), lambda j, i, be: (i, 0)),
        ],
        out_specs=pl.BlockSpec((BLK, COUT_T), lambda j, i, be: (i, j)),
    )
    return pl.pallas_call(
        _mm_body,
        grid_spec=grid_spec,
        out_shape=jax.ShapeDtypeStruct((NPAD, C_OUT), jnp.float32),
    )(block_expert, xs, experts_W, w_bcast)


@functools.cache
def _make_sc_combine():
    @functools.partial(
        pl.kernel,
        out_type=jax.ShapeDtypeStruct((T, C_OUT), jnp.float32),
        mesh=_sc_mesh(),
        scratch_types=[
            pltpu.VMEM((TCH,), jnp.int32),
            pltpu.VMEM((TCH,), jnp.int32),
            pltpu.VMEM((TCH, C_OUT), jnp.float32),
            pltpu.VMEM((TCH, C_OUT), jnp.float32),
            pltpu.SemaphoreType.DMA,
            pltpu.SemaphoreType.DMA,
        ],
    )
    def _sc_combine(ys_hbm, p1_hbm, p2_hbm, out_hbm, i1_v, i2_v, r1_v, r2_v,
                    sem1, sem2):
        wid = lax.axis_index("s") * 2 + lax.axis_index("c")
        base = wid * (T // NW)
        for ci in range(T // NW // TCH):
            off = base + ci * TCH
            pltpu.sync_copy(p1_hbm.at[pl.ds(off, TCH)], i1_v)
            pltpu.sync_copy(p2_hbm.at[pl.ds(off, TCH)], i2_v)
            c1 = pltpu.async_copy(ys_hbm.at[i1_v], r1_v, sem1)
            c2 = pltpu.async_copy(ys_hbm.at[i2_v], r2_v, sem2)
            c1.wait()
            c2.wait()
            for i in range(TCH):
                def body(j, _, i=i):
                    s = pl.ds(j * 16, 16)
                    r1_v[i, s] = r1_v[i, s] + r2_v[i, s]
                    return 0
                lax.fori_loop(0, C_OUT // 16, body, 0)
            pltpu.sync_copy(r1_v, out_hbm.at[pl.ds(off, TCH)])

    return _sc_combine


def _route_meta(i1, i2):
    e_all = jnp.concatenate([i1, i2])
    oh = (e_all[:, None] == jnp.arange(E)[None, :]).astype(jnp.float32)
    CB = 128
    NB = NPAIR // CB
    ohb = oh.reshape(NB, CB, E)
    tri = (jnp.arange(CB)[:, None] >= jnp.arange(CB)[None, :]).astype(
        jnp.float32)
    incl = jnp.einsum("lk,bke->ble", tri, ohb)
    bsum = incl[:, -1, :]
    bpre = jnp.cumsum(bsum, axis=0) - bsum
    cum = (incl + bpre[:, None, :]).reshape(NPAIR, E).astype(jnp.int32)
    counts = cum[-1]
    ohi = oh.astype(jnp.int32)
    rank = jnp.sum(ohi * cum, axis=1) - 1
    padded = ((counts + BLK - 1) // BLK) * BLK
    ends = jnp.cumsum(padded)
    starts = jnp.sum(ohi * (ends - padded)[None, :], axis=1)
    p_all = (starts + rank).astype(jnp.int32)
    block_expert = jnp.clip(
        jnp.searchsorted(ends, jnp.arange(NBLK) * BLK, side="right"),
        0, E - 1).astype(jnp.int32)
    return p_all, block_expert


def kernel(x, experts_W, gate_W, gate_b):
    b, t, c_in = x.shape
    x_flat = x.reshape(t, c_in)
    i1, i2, w1, w2 = _gate(x_flat, gate_W, gate_b)
    p_all, block_expert = _route_meta(i1, i2)
    tok_all = jnp.concatenate([jnp.arange(T, dtype=jnp.int32)] * 2)
    w_all = jnp.concatenate([w1, w2])
    slot_w = jnp.zeros((NPAD,), jnp.float32).at[p_all].set(w_all)
    w_bcast = jnp.broadcast_to(slot_w[:, None], (NPAD, 128))
    p3 = p_all.reshape(NW, NCHUNK, GCH2)
    xs = _make_sc_gather()(x_flat, tok_all, p3)
    ys = _grouped_matmul(block_expert, xs, experts_W, w_bcast)
    out = _make_sc_combine()(ys, p_all[:T], p_all[T:])
    return out.reshape(b, t, C_OUT)


# speedup vs baseline: 2.1584x; 1.0999x over previous
import functools

import jax
import jax.numpy as jnp
from jax import lax
from jax.experimental import pallas as pl
from jax.experimental.pallas import tpu as pltpu
from jax.experimental.pallas import tpu_sc as plsc

T = 2048
C_IN = 2048
C_OUT = 2048
E = 8
BLK = 256
NPAD = 2 * T + E * BLK
NBLK = NPAD // BLK
COUT_T = 2048
NCOL = C_OUT // COUT_T
ROW_T = 512

NW = 32
GCH = 32
TCH = 16


def _gate_body(x_ref, gw_ref, gb_ref, idx_ref, wts_ref):
    logits = lax.dot_general(
        x_ref[...], gw_ref[...], (((1,), (1,)), ((), ())),
        preferred_element_type=jnp.float32) + gb_ref[0, :][None, :]
    iota = lax.broadcasted_iota(jnp.int32, (ROW_T, E), 1)
    m1 = jnp.max(logits, axis=1, keepdims=True)
    i1 = jnp.min(jnp.where(logits == m1, iota, E), axis=1, keepdims=True)
    masked = jnp.where(iota == i1, -jnp.inf, logits)
    m2 = jnp.max(masked, axis=1, keepdims=True)
    i2 = jnp.min(jnp.where(masked == m2, iota, E), axis=1, keepdims=True)
    w1 = 1.0 / (1.0 + jnp.exp(m2 - m1))
    w2 = 1.0 - w1
    idx_ref[...] = jnp.where(iota == 0, i1, jnp.where(iota == 1, i2, 0))
    wts_ref[...] = jnp.where(iota == 0, w1, jnp.where(iota == 1, w2, 0.0))


def _gate(x_flat, gate_W, gate_b):
    idx, wts = pl.pallas_call(
        _gate_body,
        grid=(T // ROW_T,),
        in_specs=[
            pl.BlockSpec((ROW_T, C_IN), lambda i: (i, 0)),
            pl.BlockSpec((E, C_IN), lambda i: (0, 0)),
            pl.BlockSpec((1, E), lambda i: (0, 0)),
        ],
        out_specs=[
            pl.BlockSpec((ROW_T, E), lambda i: (i, 0)),
            pl.BlockSpec((ROW_T, E), lambda i: (i, 0)),
        ],
        out_shape=[
            jax.ShapeDtypeStruct((T, E), jnp.int32),
            jax.ShapeDtypeStruct((T, E), jnp.float32),
        ],
    )(x_flat, gate_W, gate_b.reshape(1, E))
    return idx[:, 0], idx[:, 1], wts[:, 0], wts[:, 1]


@functools.cache
def _sc_mesh():
    return plsc.VectorSubcoreMesh(core_axis_name="c", subcore_axis_name="s")


NPAIR = 2 * T


GCH2 = 16
NCHUNK = NPAIR // NW // GCH2


@functools.cache
def _make_sc_gather():
    @functools.partial(
        pl.kernel,
        out_type=jax.ShapeDtypeStruct((NPAD, C_IN), jnp.float32),
        mesh=_sc_mesh(),
        scratch_types=[
            pltpu.VMEM((NPAIR // NW,), jnp.int32),
            pltpu.VMEM((NCHUNK, GCH2), jnp.int32),
            pltpu.VMEM((GCH2, C_IN), jnp.float32),
            pltpu.VMEM((GCH2, C_IN), jnp.float32),
            pltpu.SemaphoreType.DMA,
            pltpu.SemaphoreType.DMA,
            pltpu.SemaphoreType.DMA,
            pltpu.SemaphoreType.DMA,
        ],
    )
    def _sc_gather(x_hbm, tok_hbm, pos_hbm, xs_hbm,
                   tok_v, pos_v, buf0, buf1, g0, g1, s0, s1):
        wid = lax.axis_index("s") * 2 + lax.axis_index("c")
        base = wid * (NPAIR // NW)
        pltpu.sync_copy(tok_hbm.at[pl.ds(base, NPAIR // NW)], tok_v)
        pltpu.sync_copy(pos_hbm.at[wid], pos_v)
        bufs = (buf0, buf1)
        gsems = (g0, g1)
        ssems = (s0, s1)

        def gather(ci):
            cp = pltpu.make_async_copy(
                x_hbm.at[tok_v.at[pl.ds(ci * GCH2, GCH2)]],
                bufs[ci % 2], gsems[ci % 2])
            cp.start()
            return cp

        def scatter(ci):
            cp = pltpu.make_async_copy(
                bufs[ci % 2], xs_hbm.at[pos_v.at[ci]], ssems[ci % 2])
            cp.start()
            return cp

        pend_g = {0: gather(0)}
        pend_s = {}
        for ci in range(NCHUNK):
            nxt = ci + 1
            if nxt < NCHUNK:
                if nxt - 2 >= 0:
                    pend_s.pop(nxt - 2).wait()
                pend_g[nxt] = gather(nxt)
            pend_g.pop(ci).wait()
            pend_s[ci] = scatter(ci)
        pend_s.pop(NCHUNK - 2).wait()
        pend_s.pop(NCHUNK - 1).wait()

    return _sc_gather


def _mm_body(be_ref, xs_ref, w_ref, wt_ref, ys_ref):
    acc = lax.dot_general(
        xs_ref[...], w_ref[0], (((1,), (1,)), ((), ())),
        preferred_element_type=jnp.float32)
    ys_ref[...] = acc * wt_ref[:, :1]


def _grouped_matmul(block_expert, xs, experts_W, w_bcast):
    grid_spec = pltpu.PrefetchScalarGridSpec(
        num_scalar_prefetch=1,
        grid=(NCOL, NBLK),
        in_specs=[
            pl.BlockSpec((BLK, C_IN), lambda j, i, be: (i, 0)),
            pl.BlockSpec((1, COUT_T, C_IN), lambda j, i, be: (be[i], j, 0)),
            pl.BlockSpec((BLK, 128), lambda j, i, be: (i, 0)),
        ],
        out_specs=pl.BlockSpec((BLK, COUT_T), lambda j, i, be: (i, j)),
    )
    return pl.pallas_call(
        _mm_body,
        grid_spec=grid_spec,
        out_shape=jax.ShapeDtypeStruct((NPAD, C_OUT), jnp.float32),
    )(block_expert, xs, experts_W, w_bcast)


TPW = T // NW
DCH = 8
DNCH = TPW // DCH


@functools.cache
def _make_sc_combine():
    @functools.partial(
        pl.kernel,
        out_type=jax.ShapeDtypeStruct((T, C_OUT), jnp.float32),
        mesh=_sc_mesh(),
        scratch_types=[
            pltpu.VMEM((TPW,), jnp.int32),
            pltpu.VMEM((TPW,), jnp.int32),
            pltpu.VMEM((DCH, C_OUT), jnp.float32),
            pltpu.VMEM((DCH, C_OUT), jnp.float32),
            pltpu.VMEM((DCH, C_OUT), jnp.float32),
            pltpu.VMEM((DCH, C_OUT), jnp.float32),
            pltpu.SemaphoreType.DMA,
            pltpu.SemaphoreType.DMA,
            pltpu.SemaphoreType.DMA,
            pltpu.SemaphoreType.DMA,
            pltpu.SemaphoreType.DMA,
            pltpu.SemaphoreType.DMA,
        ],
    )
    def _sc_combine(ys_hbm, p1_hbm, p2_hbm, out_hbm, p1_v, p2_v,
                    r1a, r1b, r2a, r2b, ga0, ga1, gb0, gb1, wb0, wb1):
        wid = lax.axis_index("s") * 2 + lax.axis_index("c")
        base = wid * TPW
        pltpu.sync_copy(p1_hbm.at[pl.ds(base, TPW)], p1_v)
        pltpu.sync_copy(p2_hbm.at[pl.ds(base, TPW)], p2_v)
        r1 = (r1a, r1b)
        r2 = (r2a, r2b)
        gs1 = (ga0, ga1)
        gs2 = (gb0, gb1)
        ws = (wb0, wb1)

        def gathers(ci):
            s = pl.ds(ci * DCH, DCH)
            c1 = pltpu.make_async_copy(
                ys_hbm.at[p1_v.at[s]], r1[ci % 2], gs1[ci % 2])
            c2 = pltpu.make_async_copy(
                ys_hbm.at[p2_v.at[s]], r2[ci % 2], gs2[ci % 2])
            c1.start()
            c2.start()
            return c1, c2

        def writeback(ci):
            cp = pltpu.make_async_copy(
                r1[ci % 2], out_hbm.at[pl.ds(base + ci * DCH, DCH)],
                ws[ci % 2])
            cp.start()
            return cp

        pend_g = {0: gathers(0)}
        pend_w = {}
        for ci in range(DNCH):
            nxt = ci + 1
            if nxt < DNCH:
                if nxt - 2 >= 0:
                    pend_w.pop(nxt - 2).wait()
                pend_g[nxt] = gathers(nxt)
            c1, c2 = pend_g.pop(ci)
            c1.wait()
            c2.wait()
            a = r1[ci % 2]
            b = r2[ci % 2]

            def add_row(r, _, a=a, b=b):
                for j in range(C_OUT // 16):
                    s = pl.ds(j * 16, 16)
                    a[r, s] = a[r, s] + b[r, s]
                return 0

            lax.fori_loop(0, DCH, add_row, 0)
            pend_w[ci] = writeback(ci)
        pend_w.pop(DNCH - 2).wait()
        pend_w.pop(DNCH - 1).wait()

    return _sc_combine


def _route_meta(i1, i2):
    e_all = jnp.concatenate([i1, i2])
    oh = (e_all[:, None] == jnp.arange(E)[None, :]).astype(jnp.float32)
    CB = 128
    NB = NPAIR // CB
    ohb = oh.reshape(NB, CB, E)
    tri = (jnp.arange(CB)[:, None] >= jnp.arange(CB)[None, :]).astype(
        jnp.float32)
    incl = jnp.einsum("lk,bke->ble", tri, ohb)
    bsum = incl[:, -1, :]
    bpre = jnp.cumsum(bsum, axis=0) - bsum
    cum = (incl + bpre[:, None, :]).reshape(NPAIR, E).astype(jnp.int32)
    counts = cum[-1]
    ohi = oh.astype(jnp.int32)
    rank = jnp.sum(ohi * cum, axis=1) - 1
    padded = ((counts + BLK - 1) // BLK) * BLK
    ends = jnp.cumsum(padded)
    starts = jnp.sum(ohi * (ends - padded)[None, :], axis=1)
    p_all = (starts + rank).astype(jnp.int32)
    block_expert = jnp.clip(
        jnp.searchsorted(ends, jnp.arange(NBLK) * BLK, side="right"),
        0, E - 1).astype(jnp.int32)
    return p_all, block_expert


def kernel(x, experts_W, gate_W, gate_b):
    b, t, c_in = x.shape
    x_flat = x.reshape(t, c_in)
    i1, i2, w1, w2 = _gate(x_flat, gate_W, gate_b)
    p_all, block_expert = _route_meta(i1, i2)
    tok_all = jnp.concatenate([jnp.arange(T, dtype=jnp.int32)] * 2)
    w_all = jnp.concatenate([w1, w2])
    slot_w = jnp.zeros((NPAD,), jnp.float32).at[p_all].set(w_all)
    w_bcast = jnp.broadcast_to(slot_w[:, None], (NPAD, 128))
    p3 = p_all.reshape(NW, NCHUNK, GCH2)
    xs = _make_sc_gather()(x_flat, tok_all, p3)
    ys = _grouped_matmul(block_expert, xs, experts_W, w_bcast)
    out = _make_sc_combine()(ys, p_all[:T], p_all[T:])
    return out.reshape(b, t, C_OUT)
